# baseline (speedup 1.0000x reference)
import functools

import jax
import jax.numpy as jnp
from jax import lax
from jax.experimental import pallas as pl
from jax.experimental.pallas import tpu as pltpu
from jax.experimental.pallas import tpu_sc as plsc

NC = 2
NS = 16
NW = NC * NS
IDX_CHUNK = 128


def _gather_body(n_chunks, uids, cids, utab, itab, u_out, i_out,
                 uidx, cidx, urows, irows, sem):
    wid = lax.axis_index("s") * NC + lax.axis_index("c")
    pltpu.sync_copy(uids.at[wid], uidx)
    pltpu.sync_copy(cids.at[wid], cidx)
    copies = []
    for j in range(n_chunks):
        dst = pl.ds(j * IDX_CHUNK, IDX_CHUNK)
        copies.append(pltpu.async_copy(utab.at[uidx.at[j]], urows.at[dst], sem))
        copies.append(pltpu.async_copy(itab.at[cidx.at[j]], irows.at[dst], sem))
    for cp in copies:
        cp.wait()
    pltpu.sync_copy(urows, u_out.at[wid])
    pltpu.sync_copy(irows, i_out.at[wid])


def _sc_gather(uids, cids, user_table, item_table, rows_per_w, n_chunks, d):
    mesh = plsc.VectorSubcoreMesh(core_axis_name="c", subcore_axis_name="s",
                                  num_cores=NC, num_subcores=NS)
    f = pl.kernel(
        functools.partial(_gather_body, n_chunks),
        out_type=(
            jax.ShapeDtypeStruct((NW, rows_per_w, d), jnp.bfloat16),
            jax.ShapeDtypeStruct((NW, rows_per_w, d), jnp.bfloat16),
        ),
        mesh=mesh,
        scratch_types=[
            pltpu.VMEM((n_chunks, IDX_CHUNK), jnp.int32),
            pltpu.VMEM((n_chunks, IDX_CHUNK), jnp.int32),
            pltpu.VMEM((rows_per_w, d), jnp.bfloat16),
            pltpu.VMEM((rows_per_w, d), jnp.bfloat16),
            pltpu.SemaphoreType.DMA,
        ],
        compiler_params=pltpu.CompilerParams(use_tc_tiling_on_sc=False),
    )
    return f(uids, cids, user_table, item_table)


def _mlp_body(u_ref, i_ref, w1u_ref, w1v_ref, b1_ref, w2_ref, b2_ref,
              w3_ref, b3_ref, out_ref):
    u = u_ref[...].astype(jnp.float32)
    it = i_ref[...].astype(jnp.float32)
    h = (jnp.dot(u, w1u_ref[...], preferred_element_type=jnp.float32)
         + jnp.dot(it, w1v_ref[...], preferred_element_type=jnp.float32)
         + b1_ref[...])
    h = jnp.maximum(h, 0.0)
    h = jnp.dot(h, w2_ref[...], preferred_element_type=jnp.float32) + b2_ref[...]
    h = jnp.maximum(h, 0.0)
    out_ref[...] = (jnp.sum(h * w3_ref[...], axis=1, keepdims=True)
                    + b3_ref[...])


def _mlp(u_emb, i_emb, W1u, W1v, b1, W2, b2, w3row, b3, blk):
    b, d = u_emb.shape
    h1 = W1u.shape[1]
    h2 = W2.shape[1]
    grid = (b // blk,)
    rep = lambda i: (0, 0)
    return pl.pallas_call(
        _mlp_body,
        grid=grid,
        in_specs=[
            pl.BlockSpec((blk, d), lambda i: (i, 0)),
            pl.BlockSpec((blk, d), lambda i: (i, 0)),
            pl.BlockSpec((d, h1), rep),
            pl.BlockSpec((d, h1), rep),
            pl.BlockSpec((1, h1), rep),
            pl.BlockSpec((h1, h2), rep),
            pl.BlockSpec((1, h2), rep),
            pl.BlockSpec((1, h2), rep),
            pl.BlockSpec((1, 1), rep),
        ],
        out_specs=pl.BlockSpec((blk, 1), lambda i: (i, 0)),
        out_shape=jax.ShapeDtypeStruct((b, 1), jnp.float32),
    )(u_emb, i_emb, W1u, W1v, b1, W2, b2, w3row, b3)


def kernel(user_ids, content_ids, user_table, item_table, W1, b1, W2, b2, W3, b3):
    batch = user_ids.shape[0]
    d = user_table.shape[1]
    rows_per_w = batch // NW
    n_chunks = rows_per_w // IDX_CHUNK

    uids = user_ids.astype(jnp.int32).reshape(NW, n_chunks, IDX_CHUNK)
    cids = content_ids.astype(jnp.int32).reshape(NW, n_chunks, IDX_CHUNK)

    u_emb, i_emb = _sc_gather(uids, cids,
                              user_table.astype(jnp.bfloat16),
                              item_table.astype(jnp.bfloat16),
                              rows_per_w, n_chunks, d)
    u_emb = u_emb.reshape(batch, d)
    i_emb = i_emb.reshape(batch, d)

    W1u, W1v = W1[:d, :], W1[d:, :]
    out = _mlp(u_emb, i_emb, W1u, W1v, b1.reshape(1, -1), W2,
               b2.reshape(1, -1), W3.reshape(1, -1), b3.reshape(1, 1),
               blk=2048)
    return out


# speedup vs baseline: 6.9028x; 6.9028x over previous
import functools

import jax
import jax.numpy as jnp
from jax import lax
from jax.experimental import pallas as pl
from jax.experimental.pallas import tpu as pltpu
from jax.experimental.pallas import tpu_sc as plsc

NC = 2
NS = 16
NW = NC * NS
RPW = 512
NGRP = RPW // 16
RING = 12
LOOK = 4
PROC = RING - LOOK


def _stream_table(tab, idx_v, out_v, ring, sem, v_rows):
    max_tile = (v_rows - 1) // 128

    def fire(t):
        start = jnp.minimum(t, max_tile) * 128
        pltpu.async_copy(tab.at[:, pl.ds(start, 128)], ring.at[lax.rem(t, RING)], sem)

    def wait_one():
        pltpu.make_async_copy(tab.at[:, pl.ds(0, 128)], ring.at[0], sem).wait()

    def process_pass(v, w, base):
        lo = w * 128
        hi = (w + PROC) * 128
        m = (v >= lo) & (v < hi)
        vc = jnp.clip(v, lo, hi - 1)
        t_v = lax.shift_right_logical(vc, 7)
        slot_v = lax.rem(t_v, RING)
        col_v = vc - t_v * 128
        row_idx = base + lax.iota(jnp.int32, 16)
        for row in range(32):
            row_v = jnp.full((16,), row, jnp.int32)
            got = plsc.load_gather(ring, [slot_v, row_v, col_v])
            plsc.store_scatter(out_v, [row_idx, row_v], got, mask=m)

    v0 = idx_v[0, pl.ds(0, 16)]
    w0 = lax.shift_right_logical(jnp.min(v0), 7)
    for k in range(RING):
        fire(w0 + k)
    for _ in range(RING - LOOK):
        wait_one()

    def group_body(g, w):
        j = g // 8
        c0 = (g % 8) * 16
        v = idx_v[j, pl.ds(c0, 16)]
        base = g * 16

        def not_covered(w):
            return jnp.logical_not(jnp.all(v < (w + PROC) * 128))

        def advance(w):
            process_pass(v, w, base)
            for k in range(LOOK):
                fire(w + RING + k)
            for _ in range(LOOK):
                wait_one()
            return w + LOOK

        w = lax.while_loop(not_covered, advance, w)
        process_pass(v, w, base)
        return w

    lax.fori_loop(0, NGRP, group_body, w0)
    for _ in range(LOOK):
        wait_one()


def _gather_body(v_rows, ids, tabT, out, idx_v, ring, out_v, sem):
    wid = lax.axis_index("s") * NC + lax.axis_index("c")
    pltpu.sync_copy(ids.at[wid], idx_v)
    _stream_table(tabT, idx_v, out_v, ring, sem, v_rows)
    pltpu.sync_copy(out_v, out.at[wid])


def _sc_gather(ids, tabT, d, v_rows):
    mesh = plsc.VectorSubcoreMesh(core_axis_name="c", subcore_axis_name="s",
                                  num_cores=NC, num_subcores=NS)
    f = pl.kernel(
        functools.partial(_gather_body, v_rows),
        out_type=jax.ShapeDtypeStruct((NW, RPW, d), jnp.float32),
        mesh=mesh,
        scratch_types=[
            pltpu.VMEM((RPW // 128, 128), jnp.int32),
            pltpu.VMEM((RING, d, 128), jnp.float32),
            pltpu.VMEM((RPW, d), jnp.float32),
            pltpu.SemaphoreType.DMA,
        ],
        compiler_params=pltpu.CompilerParams(use_tc_tiling_on_sc=True,
                                             disable_bounds_checks=True,
                                             needs_layout_passes=False),
    )
    return f(ids, tabT)


def _scatter_body(u_rows, i_rows, ord_u, ord_c, u_emb, i_emb,
                  uv, iv, ouv, ocv, sem):
    wid = lax.axis_index("s") * NC + lax.axis_index("c")
    pltpu.sync_copy(u_rows.at[wid], uv)
    pltpu.sync_copy(i_rows.at[wid], iv)
    pltpu.sync_copy(ord_u.at[wid], ouv)
    pltpu.sync_copy(ord_c.at[wid], ocv)
    copies = []
    for j in range(RPW // 128):
        sl = pl.ds(j * 128, 128)
        copies.append(pltpu.async_copy(uv.at[sl], u_emb.at[ouv.at[j]], sem))
        copies.append(pltpu.async_copy(iv.at[sl], i_emb.at[ocv.at[j]], sem))
    for cp in copies:
        cp.wait()


def _sc_scatter_rows(u_rows, i_rows, ord_u, ord_c, batch, d):
    mesh = plsc.VectorSubcoreMesh(core_axis_name="c", subcore_axis_name="s",
                                  num_cores=NC, num_subcores=NS)
    f = pl.kernel(
        _scatter_body,
        out_type=(
            jax.ShapeDtypeStruct((batch, d), jnp.float32),
            jax.ShapeDtypeStruct((batch, d), jnp.float32),
        ),
        mesh=mesh,
        scratch_types=[
            pltpu.VMEM((RPW, d), jnp.float32),
            pltpu.VMEM((RPW, d), jnp.float32),
            pltpu.VMEM((RPW // 128, 128), jnp.int32),
            pltpu.VMEM((RPW // 128, 128), jnp.int32),
            pltpu.SemaphoreType.DMA,
        ],
        compiler_params=pltpu.CompilerParams(use_tc_tiling_on_sc=False),
    )
    return f(u_rows, i_rows, ord_u, ord_c)


def _mlp_body(u_ref, i_ref, w1u_ref, w1v_ref, b1_ref, w2_ref, b2_ref,
              w3_ref, b3_ref, out_ref):
    h = (jnp.dot(u_ref[...], w1u_ref[...], preferred_element_type=jnp.float32)
         + jnp.dot(i_ref[...], w1v_ref[...], preferred_element_type=jnp.float32)
         + b1_ref[...])
    h = jnp.maximum(h, 0.0)
    h = jnp.dot(h, w2_ref[...], preferred_element_type=jnp.float32) + b2_ref[...]
    h = jnp.maximum(h, 0.0)
    out_ref[...] = (jnp.sum(h * w3_ref[...], axis=1, keepdims=True)
                    + b3_ref[...])


def _mlp(u_emb, i_emb, W1u, W1v, b1, W2, b2, w3row, b3, blk):
    b, d = u_emb.shape
    h1 = W1u.shape[1]
    h2 = W2.shape[1]
    grid = (b // blk,)
    rep = lambda i: (0, 0)
    return pl.pallas_call(
        _mlp_body,
        grid=grid,
        in_specs=[
            pl.BlockSpec((blk, d), lambda i: (i, 0)),
            pl.BlockSpec((blk, d), lambda i: (i, 0)),
            pl.BlockSpec((d, h1), rep),
            pl.BlockSpec((d, h1), rep),
            pl.BlockSpec((1, h1), rep),
            pl.BlockSpec((h1, h2), rep),
            pl.BlockSpec((1, h2), rep),
            pl.BlockSpec((1, h2), rep),
            pl.BlockSpec((1, 1), rep),
        ],
        out_specs=pl.BlockSpec((blk, 1), lambda i: (i, 0)),
        out_shape=jax.ShapeDtypeStruct((b, 1), jnp.float32),
    )(u_emb, i_emb, W1u, W1v, b1, W2, b2, w3row, b3)


def kernel(user_ids, content_ids, user_table, item_table, W1, b1, W2, b2, W3, b3):
    batch = user_ids.shape[0]
    d = user_table.shape[1]
    v_rows = user_table.shape[0]

    pos = jnp.arange(batch, dtype=jnp.int32)
    su, ord_u = jax.lax.sort_key_val(user_ids.astype(jnp.int32), pos)
    sc, ord_c = jax.lax.sort_key_val(content_ids.astype(jnp.int32), pos)

    u_rows = _sc_gather(su.reshape(NW, RPW // 128, 128), user_table.T,
                        d, v_rows)
    i_rows = _sc_gather(sc.reshape(NW, RPW // 128, 128), item_table.T,
                        d, v_rows)

    u_emb, i_emb = _sc_scatter_rows(u_rows, i_rows,
                                    ord_u.reshape(NW, RPW // 128, 128),
                                    ord_c.reshape(NW, RPW // 128, 128),
                                    batch, d)

    W1u, W1v = W1[:d, :], W1[d:, :]
    return _mlp(u_emb, i_emb, W1u, W1v, b1.reshape(1, -1), W2,
                b2.reshape(1, -1), W3.reshape(1, -1), b3.reshape(1, 1),
                blk=2048)


# speedup vs baseline: 7.3358x; 1.0627x over previous
import functools

import jax
import jax.numpy as jnp
from jax import lax
from jax.experimental import pallas as pl
from jax.experimental.pallas import tpu as pltpu
from jax.experimental.pallas import tpu_sc as plsc

NC = 2
NS = 16
NW = NC * NS
RPW = 512
NGRP = RPW // 16
RING = 12
LOOK = 4
PROC = RING - LOOK


def _stream_table(tab, idx_v, out_v, ring, sem, v_rows):
    max_tile = (v_rows - 1) // 128

    def fire(t):
        start = jnp.minimum(t, max_tile) * 128
        pltpu.async_copy(tab.at[:, pl.ds(start, 128)], ring.at[lax.rem(t, RING)], sem)

    def wait_one():
        pltpu.make_async_copy(tab.at[:, pl.ds(0, 128)], ring.at[0], sem).wait()

    def process_pass(v, w, base):
        lo = w * 128
        hi = (w + PROC) * 128
        m = (v >= lo) & (v < hi)
        vc = jnp.clip(v, lo, hi - 1)
        t_v = lax.shift_right_logical(vc, 7)
        slot_v = lax.rem(t_v, RING)
        col_v = vc - t_v * 128
        row_idx = base + lax.iota(jnp.int32, 16)
        for row in range(32):
            row_v = jnp.full((16,), row, jnp.int32)
            got = plsc.load_gather(ring, [slot_v, row_v, col_v])
            flat = row_idx * 32 + row
            plsc.store_scatter(out_v,
                               [lax.shift_right_logical(flat, 7), flat & 127],
                               got, mask=m)

    v0 = idx_v[0, pl.ds(0, 16)]
    w0 = lax.shift_right_logical(jnp.min(v0), 7)
    for k in range(RING):
        fire(w0 + k)
    for _ in range(RING - LOOK):
        wait_one()

    def group_body(g, w):
        j = g // 8
        c0 = (g % 8) * 16
        v = idx_v[j, pl.ds(c0, 16)]
        base = g * 16

        def not_covered(w):
            return jnp.logical_not(jnp.all(v < (w + PROC) * 128))

        def advance(w):
            process_pass(v, w, base)
            for k in range(LOOK):
                fire(w + RING + k)
            for _ in range(LOOK):
                wait_one()
            return w + LOOK

        w = lax.while_loop(not_covered, advance, w)
        process_pass(v, w, base)
        return w

    lax.fori_loop(0, NGRP, group_body, w0)
    for _ in range(LOOK):
        wait_one()


def _gather_body(v_rows, ids, tabT, out, idx_v, ring, out_v, sem):
    wid = lax.axis_index("s") * NC + lax.axis_index("c")
    pltpu.sync_copy(ids.at[wid], idx_v)
    _stream_table(tabT, idx_v, out_v, ring, sem, v_rows)
    pltpu.sync_copy(out_v, out.at[wid])


def _sc_gather(ids, tabT, d, v_rows):
    mesh = plsc.VectorSubcoreMesh(core_axis_name="c", subcore_axis_name="s",
                                  num_cores=NC, num_subcores=NS)
    f = pl.kernel(
        functools.partial(_gather_body, v_rows),
        out_type=jax.ShapeDtypeStruct((NW, RPW * d // 128, 128), jnp.float32),
        mesh=mesh,
        scratch_types=[
            pltpu.VMEM((RPW // 128, 128), jnp.int32),
            pltpu.VMEM((RING, d, 128), jnp.float32),
            pltpu.VMEM((RPW * d // 128, 128), jnp.float32),
            pltpu.SemaphoreType.DMA,
        ],
        compiler_params=pltpu.CompilerParams(use_tc_tiling_on_sc=True,
                                             disable_bounds_checks=True,
                                             needs_layout_passes=False),
    )
    return f(ids, tabT)


def _scatter_body(u_rows, i_rows, ord_u, ord_c, u_emb, i_emb,
                  uv, iv, ouv, ocv, sem):
    wid = lax.axis_index("s") * NC + lax.axis_index("c")
    pltpu.sync_copy(u_rows.at[wid], uv)
    pltpu.sync_copy(i_rows.at[wid], iv)
    pltpu.sync_copy(ord_u.at[wid], ouv)
    pltpu.sync_copy(ord_c.at[wid], ocv)
    copies = []
    for j in range(RPW // 128):
        sl = pl.ds(j * 128, 128)
        copies.append(pltpu.async_copy(uv.at[sl], u_emb.at[ouv.at[j]], sem))
        copies.append(pltpu.async_copy(iv.at[sl], i_emb.at[ocv.at[j]], sem))
    for cp in copies:
        cp.wait()


def _sc_scatter_rows(u_rows, i_rows, ord_u, ord_c, batch, d):
    mesh = plsc.VectorSubcoreMesh(core_axis_name="c", subcore_axis_name="s",
                                  num_cores=NC, num_subcores=NS)
    f = pl.kernel(
        _scatter_body,
        out_type=(
            jax.ShapeDtypeStruct((batch, d), jnp.float32),
            jax.ShapeDtypeStruct((batch, d), jnp.float32),
        ),
        mesh=mesh,
        scratch_types=[
            pltpu.VMEM((RPW, d), jnp.float32),
            pltpu.VMEM((RPW, d), jnp.float32),
            pltpu.VMEM((RPW // 128, 128), jnp.int32),
            pltpu.VMEM((RPW // 128, 128), jnp.int32),
            pltpu.SemaphoreType.DMA,
        ],
        compiler_params=pltpu.CompilerParams(use_tc_tiling_on_sc=False),
    )
    return f(u_rows, i_rows, ord_u, ord_c)


def _mlp_body(u_ref, i_ref, w1u_ref, w1v_ref, b1_ref, w2_ref, b2_ref,
              w3_ref, b3_ref, out_ref):
    bf = jnp.bfloat16
    h = (jnp.dot(u_ref[...].astype(bf), w1u_ref[...].astype(bf),
                 preferred_element_type=jnp.float32)
         + jnp.dot(i_ref[...].astype(bf), w1v_ref[...].astype(bf),
                   preferred_element_type=jnp.float32)
         + b1_ref[...])
    h = jnp.maximum(h, 0.0)
    h = jnp.dot(h.astype(bf), w2_ref[...].astype(bf),
                preferred_element_type=jnp.float32) + b2_ref[...]
    h = jnp.maximum(h, 0.0)
    out_ref[...] = (jnp.sum(h * w3_ref[...], axis=1, keepdims=True)
                    + b3_ref[...])


def _mlp(u_emb, i_emb, W1u, W1v, b1, W2, b2, w3row, b3, blk):
    b, d = u_emb.shape
    h1 = W1u.shape[1]
    h2 = W2.shape[1]
    grid = (b // blk,)
    rep = lambda i: (0, 0)
    return pl.pallas_call(
        _mlp_body,
        grid=grid,
        in_specs=[
            pl.BlockSpec((blk, d), lambda i: (i, 0)),
            pl.BlockSpec((blk, d), lambda i: (i, 0)),
            pl.BlockSpec((d, h1), rep),
            pl.BlockSpec((d, h1), rep),
            pl.BlockSpec((1, h1), rep),
            pl.BlockSpec((h1, h2), rep),
            pl.BlockSpec((1, h2), rep),
            pl.BlockSpec((1, h2), rep),
            pl.BlockSpec((1, 1), rep),
        ],
        out_specs=pl.BlockSpec((blk, 1), lambda i: (i, 0)),
        out_shape=jax.ShapeDtypeStruct((b, 1), jnp.float32),
    )(u_emb, i_emb, W1u, W1v, b1, W2, b2, w3row, b3)


def kernel(user_ids, content_ids, user_table, item_table, W1, b1, W2, b2, W3, b3):
    batch = user_ids.shape[0]
    d = user_table.shape[1]
    v_rows = user_table.shape[0]

    pos = jnp.arange(batch, dtype=jnp.int32)
    su, ord_u = jax.lax.sort_key_val(user_ids.astype(jnp.int32), pos)
    sc, ord_c = jax.lax.sort_key_val(content_ids.astype(jnp.int32), pos)

    u_rows = _sc_gather(su.reshape(NW, RPW // 128, 128), user_table.T,
                        d, v_rows).reshape(NW, RPW, d)
    i_rows = _sc_gather(sc.reshape(NW, RPW // 128, 128), item_table.T,
                        d, v_rows).reshape(NW, RPW, d)

    u_emb, i_emb = _sc_scatter_rows(u_rows, i_rows,
                                    ord_u.reshape(NW, RPW // 128, 128),
                                    ord_c.reshape(NW, RPW // 128, 128),
                                    batch, d)

    W1u, W1v = W1[:d, :], W1[d:, :]
    return _mlp(u_emb, i_emb, W1u, W1v, b1.reshape(1, -1), W2,
                b2.reshape(1, -1), W3.reshape(1, -1), b3.reshape(1, 1),
                blk=4096)


# speedup vs baseline: 7.9137x; 1.0788x over previous
import functools

import jax
import jax.numpy as jnp
from jax import lax
from jax.experimental import pallas as pl
from jax.experimental.pallas import tpu as pltpu
from jax.experimental.pallas import tpu_sc as plsc

NC = 2
NS = 16
NW = NC * NS
RPW = 512
NGRP = RPW // 16
RING = 12
LOOK = 4
PROC = RING - LOOK


def _stream_table(tab, idx_v, out_v, ring, sem, v_rows):
    max_tile = (v_rows - 1) // 128

    def fire(t):
        start = jnp.minimum(t, max_tile) * 128
        pltpu.async_copy(tab.at[:, pl.ds(start, 128)], ring.at[lax.rem(t, RING)], sem)

    def wait_one():
        pltpu.make_async_copy(tab.at[:, pl.ds(0, 128)], ring.at[0], sem).wait()

    def process_pass(v, w, base):
        lo = w * 128
        hi = (w + PROC) * 128
        m = (v >= lo) & (v < hi)
        vc = jnp.clip(v, lo, hi - 1)
        t_v = lax.shift_right_logical(vc, 7)
        slot_v = lax.rem(t_v, RING)
        col_v = vc - t_v * 128
        row_idx = base + lax.iota(jnp.int32, 16)
        for row in range(32):
            row_v = jnp.full((16,), row, jnp.int32)
            got = plsc.load_gather(ring, [slot_v, row_v, col_v])
            flat = row_idx * 32 + row
            plsc.store_scatter(out_v,
                               [lax.shift_right_logical(flat, 7), flat & 127],
                               got, mask=m)

    v0 = idx_v[0, pl.ds(0, 16)]
    w0 = lax.shift_right_logical(jnp.min(v0), 7)
    for k in range(RING):
        fire(w0 + k)
    for _ in range(RING - LOOK):
        wait_one()

    def group_body(g, w):
        j = g // 8
        c0 = (g % 8) * 16
        v = idx_v[j, pl.ds(c0, 16)]
        base = g * 16

        def not_covered(w):
            return jnp.logical_not(jnp.all(v < (w + PROC) * 128))

        def advance(w):
            process_pass(v, w, base)
            for k in range(LOOK):
                fire(w + RING + k)
            for _ in range(LOOK):
                wait_one()
            return w + LOOK

        w = lax.while_loop(not_covered, advance, w)
        process_pass(v, w, base)
        return w

    lax.fori_loop(0, NGRP, group_body, w0)
    for _ in range(LOOK):
        wait_one()


def _gather_body(v_rows, ids, tabT, out, idx_v, ring, out_v, sem):
    wid = lax.axis_index("s") * NC + lax.axis_index("c")
    pltpu.sync_copy(ids.at[wid], idx_v)
    _stream_table(tabT, idx_v, out_v, ring, sem, v_rows)
    pltpu.sync_copy(out_v, out.at[wid])


def _sc_gather(ids, tabT, d, v_rows):
    mesh = plsc.VectorSubcoreMesh(core_axis_name="c", subcore_axis_name="s",
                                  num_cores=NC, num_subcores=NS)
    f = pl.kernel(
        functools.partial(_gather_body, v_rows),
        out_type=jax.ShapeDtypeStruct((NW, RPW * d // 128, 128), jnp.float32),
        mesh=mesh,
        scratch_types=[
            pltpu.VMEM((RPW // 128, 128), jnp.int32),
            pltpu.VMEM((RING, d, 128), jnp.float32),
            pltpu.VMEM((RPW * d // 128, 128), jnp.float32),
            pltpu.SemaphoreType.DMA,
        ],
        compiler_params=pltpu.CompilerParams(use_tc_tiling_on_sc=True,
                                             disable_bounds_checks=True,
                                             needs_layout_passes=False),
    )
    return f(ids, tabT)


def _scatter_body(u_rows, i_rows, ord_u, ord_c, u_emb, i_emb,
                  uv, iv, ouv, ocv, sem):
    wid = lax.axis_index("s") * NC + lax.axis_index("c")
    pltpu.sync_copy(u_rows.at[wid], uv)
    pltpu.sync_copy(i_rows.at[wid], iv)
    pltpu.sync_copy(ord_u.at[wid], ouv)
    pltpu.sync_copy(ord_c.at[wid], ocv)
    copies = []
    for j in range(RPW // 128):
        sl = pl.ds(j * 128, 128)
        copies.append(pltpu.async_copy(uv.at[sl], u_emb.at[ouv.at[j]], sem))
        copies.append(pltpu.async_copy(iv.at[sl], i_emb.at[ocv.at[j]], sem))
    for cp in copies:
        cp.wait()


def _sc_scatter_rows(u_rows, i_rows, ord_u, ord_c, batch, d):
    mesh = plsc.VectorSubcoreMesh(core_axis_name="c", subcore_axis_name="s",
                                  num_cores=NC, num_subcores=NS)
    f = pl.kernel(
        _scatter_body,
        out_type=(
            jax.ShapeDtypeStruct((batch, d), jnp.float32),
            jax.ShapeDtypeStruct((batch, d), jnp.float32),
        ),
        mesh=mesh,
        scratch_types=[
            pltpu.VMEM((RPW, d), jnp.float32),
            pltpu.VMEM((RPW, d), jnp.float32),
            pltpu.VMEM((RPW // 128, 128), jnp.int32),
            pltpu.VMEM((RPW // 128, 128), jnp.int32),
            pltpu.SemaphoreType.DMA,
        ],
        compiler_params=pltpu.CompilerParams(use_tc_tiling_on_sc=False),
    )
    return f(u_rows, i_rows, ord_u, ord_c)


def _mlp_body(u_ref, i_ref, w1u_ref, w1v_ref, b1_ref, w2_ref, b2_ref,
              w3_ref, b3_ref, out_ref):
    bf = jnp.bfloat16
    h = (jnp.dot(u_ref[...].astype(bf), w1u_ref[...].astype(bf),
                 preferred_element_type=jnp.float32)
         + jnp.dot(i_ref[...].astype(bf), w1v_ref[...].astype(bf),
                   preferred_element_type=jnp.float32)
         + b1_ref[...])
    h = jnp.maximum(h, 0.0)
    h = jnp.dot(h.astype(bf), w2_ref[...].astype(bf),
                preferred_element_type=jnp.float32) + b2_ref[...]
    h = jnp.maximum(h, 0.0)
    out_ref[...] = (jnp.dot(h.astype(bf), w3_ref[...].astype(bf),
                            preferred_element_type=jnp.float32)
                    + b3_ref[...])


def _mlp(u_pk, i_pk, U4, V4, b1_4, W2_4, b2_4, W3_4, b3, blkp):
    bp, dpk = u_pk.shape
    h1p = U4.shape[1]
    h2p = W2_4.shape[1]
    pk = W3_4.shape[1]
    grid = (bp // blkp,)
    rep = lambda i: (0, 0)
    return pl.pallas_call(
        _mlp_body,
        grid=grid,
        in_specs=[
            pl.BlockSpec((blkp, dpk), lambda i: (i, 0)),
            pl.BlockSpec((blkp, dpk), lambda i: (i, 0)),
            pl.BlockSpec((dpk, h1p), rep),
            pl.BlockSpec((dpk, h1p), rep),
            pl.BlockSpec((1, h1p), rep),
            pl.BlockSpec((h1p, h2p), rep),
            pl.BlockSpec((1, h2p), rep),
            pl.BlockSpec((h2p, pk), rep),
            pl.BlockSpec((1, 1), rep),
        ],
        out_specs=pl.BlockSpec((blkp, pk), lambda i: (i, 0)),
        out_shape=jax.ShapeDtypeStruct((bp, pk), jnp.float32),
    )(u_pk, i_pk, U4, V4, b1_4, W2_4, b2_4, W3_4, b3)


def kernel(user_ids, content_ids, user_table, item_table, W1, b1, W2, b2, W3, b3):
    batch = user_ids.shape[0]
    d = user_table.shape[1]
    v_rows = user_table.shape[0]

    pos = jnp.arange(batch, dtype=jnp.int32)
    su, ord_u = jax.lax.sort_key_val(user_ids.astype(jnp.int32), pos)
    sc, ord_c = jax.lax.sort_key_val(content_ids.astype(jnp.int32), pos)

    u_rows = _sc_gather(su.reshape(NW, RPW // 128, 128), user_table.T,
                        d, v_rows).reshape(NW, RPW, d)
    i_rows = _sc_gather(sc.reshape(NW, RPW // 128, 128), item_table.T,
                        d, v_rows).reshape(NW, RPW, d)

    u_emb, i_emb = _sc_scatter_rows(u_rows, i_rows,
                                    ord_u.reshape(NW, RPW // 128, 128),
                                    ord_c.reshape(NW, RPW // 128, 128),
                                    batch, d)

    W1u, W1v = W1[:d, :], W1[d:, :]
    pk = 128 // d
    eye = jnp.eye(pk, dtype=jnp.float32)
    U4 = jnp.kron(eye, W1u)
    V4 = jnp.kron(eye, W1v)
    W2_4 = jnp.kron(eye, W2)
    W3_4 = jnp.kron(eye, W3)
    b1_4 = jnp.tile(b1, pk).reshape(1, -1)
    b2_4 = jnp.tile(b2, pk).reshape(1, -1)
    out = _mlp(u_emb.reshape(batch // pk, 128), i_emb.reshape(batch // pk, 128),
               U4, V4, b1_4, W2_4, b2_4, W3_4, b3.reshape(1, 1),
               blkp=1024)
    return out.reshape(batch, 1)


# speedup vs baseline: 8.7037x; 1.0998x over previous
import functools

import jax
import jax.numpy as jnp
from jax import lax
from jax.experimental import pallas as pl
from jax.experimental.pallas import tpu as pltpu
from jax.experimental.pallas import tpu_sc as plsc

NC = 2
NS = 16
NW = NC * NS
RPW = 512
NGRP = RPW // 16
RING = 18
LOOK = 6
PROC = RING - LOOK


def _stream_table(tab, idx_v, out_v, ring, sem, v_rows):
    max_tile = (v_rows - 1) // 128

    def fire(t):
        start = jnp.minimum(t, max_tile) * 128
        pltpu.async_copy(tab.at[:, pl.ds(start, 128)], ring.at[lax.rem(t, RING)], sem)

    def wait_one():
        pltpu.make_async_copy(tab.at[:, pl.ds(0, 128)], ring.at[0], sem).wait()

    def process_pass(v, w, base):
        lo = w * 128
        hi = (w + PROC) * 128
        m = (v >= lo) & (v < hi)
        vc = jnp.clip(v, lo, hi - 1)
        t_v = lax.shift_right_logical(vc, 7)
        slot_v = lax.rem(t_v, RING)
        col_v = vc - t_v * 128
        row_idx = base + lax.iota(jnp.int32, 16)
        for row in range(32):
            row_v = jnp.full((16,), row, jnp.int32)
            got = plsc.load_gather(ring, [slot_v, row_v, col_v])
            flat = row_idx * 32 + row
            plsc.store_scatter(out_v,
                               [lax.shift_right_logical(flat, 7), flat & 127],
                               got, mask=m)

    v0 = idx_v[0, pl.ds(0, 16)]
    w0 = lax.shift_right_logical(jnp.min(v0), 7)
    for k in range(RING):
        fire(w0 + k)
    for _ in range(RING - LOOK):
        wait_one()

    def group_body(g, w):
        j = g // 8
        c0 = (g % 8) * 16
        v = idx_v[j, pl.ds(c0, 16)]
        base = g * 16

        def not_covered(w):
            return jnp.logical_not(jnp.all(v < (w + PROC) * 128))

        def advance(w):
            process_pass(v, w, base)
            for k in range(LOOK):
                fire(w + RING + k)
            for _ in range(LOOK):
                wait_one()
            return w + LOOK

        w = lax.while_loop(not_covered, advance, w)
        process_pass(v, w, base)
        return w

    lax.fori_loop(0, NGRP, group_body, w0)
    for _ in range(LOOK):
        wait_one()


def _gather_body(v_rows, ids, tabT, out, idx_v, ring, out_v, sem):
    wid = lax.axis_index("s") * NC + lax.axis_index("c")
    pltpu.sync_copy(ids.at[wid], idx_v)
    _stream_table(tabT, idx_v, out_v, ring, sem, v_rows)
    pltpu.sync_copy(out_v, out.at[wid])


def _sc_gather(ids, tabT, d, v_rows):
    mesh = plsc.VectorSubcoreMesh(core_axis_name="c", subcore_axis_name="s",
                                  num_cores=NC, num_subcores=NS)
    f = pl.kernel(
        functools.partial(_gather_body, v_rows),
        out_type=jax.ShapeDtypeStruct((NW, RPW * d // 128, 128), jnp.float32),
        mesh=mesh,
        scratch_types=[
            pltpu.VMEM((RPW // 128, 128), jnp.int32),
            pltpu.VMEM((RING, d, 128), jnp.float32),
            pltpu.VMEM((RPW * d // 128, 128), jnp.float32),
            pltpu.SemaphoreType.DMA,
        ],
        compiler_params=pltpu.CompilerParams(use_tc_tiling_on_sc=True,
                                             disable_bounds_checks=True,
                                             needs_layout_passes=False),
    )
    return f(ids, tabT)


def _scatter_body(u_rows, i_rows, ord_u, ord_c, u_emb, i_emb,
                  uv, iv, ouv, ocv, sem):
    wid = lax.axis_index("s") * NC + lax.axis_index("c")
    pltpu.sync_copy(u_rows.at[wid], uv)
    pltpu.sync_copy(i_rows.at[wid], iv)
    pltpu.sync_copy(ord_u.at[wid], ouv)
    pltpu.sync_copy(ord_c.at[wid], ocv)
    copies = []
    for j in range(RPW // 128):
        sl = pl.ds(j * 128, 128)
        copies.append(pltpu.async_copy(uv.at[sl], u_emb.at[ouv.at[j]], sem))
        copies.append(pltpu.async_copy(iv.at[sl], i_emb.at[ocv.at[j]], sem))
    for cp in copies:
        cp.wait()


def _sc_scatter_rows(u_rows, i_rows, ord_u, ord_c, batch, d):
    mesh = plsc.VectorSubcoreMesh(core_axis_name="c", subcore_axis_name="s",
                                  num_cores=NC, num_subcores=NS)
    f = pl.kernel(
        _scatter_body,
        out_type=(
            jax.ShapeDtypeStruct((batch, d), jnp.float32),
            jax.ShapeDtypeStruct((batch, d), jnp.float32),
        ),
        mesh=mesh,
        scratch_types=[
            pltpu.VMEM((RPW, d), jnp.float32),
            pltpu.VMEM((RPW, d), jnp.float32),
            pltpu.VMEM((RPW // 128, 128), jnp.int32),
            pltpu.VMEM((RPW // 128, 128), jnp.int32),
            pltpu.SemaphoreType.DMA,
        ],
        compiler_params=pltpu.CompilerParams(use_tc_tiling_on_sc=False),
    )
    return f(u_rows, i_rows, ord_u, ord_c)


def _mlp_body(u_ref, i_ref, w1u_ref, w1v_ref, b1_ref, w2_ref, b2_ref,
              w3_ref, b3_ref, out_ref):
    bf = jnp.bfloat16
    h = (jnp.dot(u_ref[...].astype(bf), w1u_ref[...],
                 preferred_element_type=jnp.float32)
         + jnp.dot(i_ref[...].astype(bf), w1v_ref[...],
                   preferred_element_type=jnp.float32)
         + b1_ref[...])
    h = jnp.maximum(h, 0.0)
    h = jnp.dot(h.astype(bf), w2_ref[...],
                preferred_element_type=jnp.float32) + b2_ref[...]
    h = jnp.maximum(h, 0.0)
    out_ref[...] = (jnp.dot(h.astype(bf), w3_ref[...],
                            preferred_element_type=jnp.float32)
                    + b3_ref[...])


def _mlp(u_pk, i_pk, U4, V4, b1_4, W2_4, b2_4, W3_4, b3, blkp):
    bp, dpk = u_pk.shape
    h1p = U4.shape[1]
    h2p = W2_4.shape[1]
    pk = W3_4.shape[1]
    grid = (bp // blkp,)
    rep = lambda i: (0, 0)
    return pl.pallas_call(
        _mlp_body,
        grid=grid,
        in_specs=[
            pl.BlockSpec((blkp, dpk), lambda i: (i, 0)),
            pl.BlockSpec((blkp, dpk), lambda i: (i, 0)),
            pl.BlockSpec((dpk, h1p), rep),
            pl.BlockSpec((dpk, h1p), rep),
            pl.BlockSpec((1, h1p), rep),
            pl.BlockSpec((h1p, h2p), rep),
            pl.BlockSpec((1, h2p), rep),
            pl.BlockSpec((h2p, pk), rep),
            pl.BlockSpec((1, 1), rep),
        ],
        out_specs=pl.BlockSpec((blkp, pk), lambda i: (i, 0)),
        out_shape=jax.ShapeDtypeStruct((bp, pk), jnp.float32),
    )(u_pk, i_pk, U4, V4, b1_4, W2_4, b2_4, W3_4, b3)


def kernel(user_ids, content_ids, user_table, item_table, W1, b1, W2, b2, W3, b3):
    batch = user_ids.shape[0]
    d = user_table.shape[1]
    v_rows = user_table.shape[0]

    pos = jnp.arange(batch, dtype=jnp.int32)
    su, ord_u = jax.lax.sort_key_val(user_ids.astype(jnp.int32), pos)
    sc, ord_c = jax.lax.sort_key_val(content_ids.astype(jnp.int32), pos)

    u_rows = _sc_gather(su.reshape(NW, RPW // 128, 128), user_table.T,
                        d, v_rows).reshape(NW, RPW, d)
    i_rows = _sc_gather(sc.reshape(NW, RPW // 128, 128), item_table.T,
                        d, v_rows).reshape(NW, RPW, d)

    u_emb, i_emb = _sc_scatter_rows(u_rows, i_rows,
                                    ord_u.reshape(NW, RPW // 128, 128),
                                    ord_c.reshape(NW, RPW // 128, 128),
                                    batch, d)

    W1u, W1v = W1[:d, :], W1[d:, :]
    pk = 128 // d
    eye = jnp.eye(pk, dtype=jnp.float32)
    bf = jnp.bfloat16
    U4 = jnp.kron(eye, W1u).astype(bf)
    V4 = jnp.kron(eye, W1v).astype(bf)
    W2_4 = jnp.kron(eye, W2).astype(bf)
    W3_4 = jnp.kron(eye, W3).astype(bf)
    b1_4 = jnp.tile(b1, pk).reshape(1, -1)
    b2_4 = jnp.tile(b2, pk).reshape(1, -1)
    out = _mlp(u_emb.reshape(batch // pk, 128), i_emb.reshape(batch // pk, 128),
               U4, V4, b1_4, W2_4, b2_4, W3_4, b3.reshape(1, 1),
               blkp=2048)
    return out.reshape(batch, 1)


# speedup vs baseline: 8.9718x; 1.0308x over previous
import functools

import jax
import jax.numpy as jnp
from jax import lax
from jax.experimental import pallas as pl
from jax.experimental.pallas import tpu as pltpu
from jax.experimental.pallas import tpu_sc as plsc

NC = 2
NS = 16
NW = NC * NS
RPW = 512
NGRP = RPW // 16
RING = 19
LOOK = 7
PROC = RING - LOOK


def _stream_table(tab, idx_v, out_v, ring, sem, v_rows):
    max_tile = (v_rows - 1) // 128

    def fire(t):
        start = jnp.minimum(t, max_tile) * 128
        pltpu.async_copy(tab.at[:, pl.ds(start, 128)], ring.at[lax.rem(t, RING)], sem)

    def wait_one():
        pltpu.make_async_copy(tab.at[:, pl.ds(0, 128)], ring.at[0], sem).wait()

    def process_pass(v, w, base):
        lo = w * 128
        hi = (w + PROC) * 128
        m = (v >= lo) & (v < hi)
        vc = jnp.clip(v, lo, hi - 1)
        t_v = lax.shift_right_logical(vc, 7)
        slot_v = lax.rem(t_v, RING)
        col_v = vc - t_v * 128
        row_idx = base + lax.iota(jnp.int32, 16)
        for row in range(32):
            row_v = jnp.full((16,), row, jnp.int32)
            got = plsc.load_gather(ring, [slot_v, row_v, col_v])
            flat = row_idx * 32 + row
            plsc.store_scatter(out_v,
                               [lax.shift_right_logical(flat, 7), flat & 127],
                               got, mask=m)

    v0 = idx_v[0, pl.ds(0, 16)]
    w0 = lax.shift_right_logical(jnp.min(v0), 7)
    for k in range(RING):
        fire(w0 + k)
    for _ in range(RING - LOOK):
        wait_one()

    def group_body(g, w):
        j = g // 8
        c0 = (g % 8) * 16
        v = idx_v[j, pl.ds(c0, 16)]
        base = g * 16

        def not_covered(w):
            return jnp.logical_not(jnp.all(v < (w + PROC) * 128))

        def advance(w):
            process_pass(v, w, base)
            for k in range(LOOK):
                fire(w + RING + k)
            for _ in range(LOOK):
                wait_one()
            return w + LOOK

        w = lax.while_loop(not_covered, advance, w)
        process_pass(v, w, base)
        return w

    lax.fori_loop(0, NGRP, group_body, w0)
    for _ in range(LOOK):
        wait_one()


def _gather_body(v_rows, ids, tabT, out, idx_v, ring, out_v, sem):
    wid = lax.axis_index("s") * NC + lax.axis_index("c")
    pltpu.sync_copy(ids.at[wid], idx_v)
    _stream_table(tabT, idx_v, out_v, ring, sem, v_rows)
    pltpu.sync_copy(out_v, out.at[wid])


def _sc_gather(ids, tabT, d, v_rows):
    mesh = plsc.VectorSubcoreMesh(core_axis_name="c", subcore_axis_name="s",
                                  num_cores=NC, num_subcores=NS)
    f = pl.kernel(
        functools.partial(_gather_body, v_rows),
        out_type=jax.ShapeDtypeStruct((NW, RPW * d // 128, 128), jnp.float32),
        mesh=mesh,
        scratch_types=[
            pltpu.VMEM((RPW // 128, 128), jnp.int32),
            pltpu.VMEM((RING, d, 128), jnp.float32),
            pltpu.VMEM((RPW * d // 128, 128), jnp.float32),
            pltpu.SemaphoreType.DMA,
        ],
        compiler_params=pltpu.CompilerParams(use_tc_tiling_on_sc=True,
                                             disable_bounds_checks=True,
                                             needs_layout_passes=False),
    )
    return f(ids, tabT)


def _scatter_body(u_rows, i_rows, ord_u, ord_c, u_emb, i_emb,
                  uv, iv, ouv, ocv, sem):
    wid = lax.axis_index("s") * NC + lax.axis_index("c")
    pltpu.sync_copy(u_rows.at[wid], uv)
    pltpu.sync_copy(i_rows.at[wid], iv)
    pltpu.sync_copy(ord_u.at[wid], ouv)
    pltpu.sync_copy(ord_c.at[wid], ocv)
    copies = []
    for j in range(RPW // 128):
        sl = pl.ds(j * 128, 128)
        copies.append(pltpu.async_copy(uv.at[sl], u_emb.at[ouv.at[j]], sem))
        copies.append(pltpu.async_copy(iv.at[sl], i_emb.at[ocv.at[j]], sem))
    for cp in copies:
        cp.wait()


def _sc_scatter_rows(u_rows, i_rows, ord_u, ord_c, batch, d):
    mesh = plsc.VectorSubcoreMesh(core_axis_name="c", subcore_axis_name="s",
                                  num_cores=NC, num_subcores=NS)
    f = pl.kernel(
        _scatter_body,
        out_type=(
            jax.ShapeDtypeStruct((batch, d), jnp.float32),
            jax.ShapeDtypeStruct((batch, d), jnp.float32),
        ),
        mesh=mesh,
        scratch_types=[
            pltpu.VMEM((RPW, d), jnp.float32),
            pltpu.VMEM((RPW, d), jnp.float32),
            pltpu.VMEM((RPW // 128, 128), jnp.int32),
            pltpu.VMEM((RPW // 128, 128), jnp.int32),
            pltpu.SemaphoreType.DMA,
        ],
        compiler_params=pltpu.CompilerParams(use_tc_tiling_on_sc=False),
    )
    return f(u_rows, i_rows, ord_u, ord_c)


def _mlp_body(u_ref, i_ref, w1u_ref, w1v_ref, b1_ref, w2_ref, b2_ref,
              w3_ref, b3_ref, out_ref):
    bf = jnp.bfloat16
    h = (jnp.dot(u_ref[...].astype(bf), w1u_ref[...],
                 preferred_element_type=jnp.float32)
         + jnp.dot(i_ref[...].astype(bf), w1v_ref[...],
                   preferred_element_type=jnp.float32)
         + b1_ref[...])
    h = jnp.maximum(h, 0.0)
    h = jnp.dot(h.astype(bf), w2_ref[...],
                preferred_element_type=jnp.float32) + b2_ref[...]
    h = jnp.maximum(h, 0.0)
    out_ref[...] = (jnp.dot(h.astype(bf), w3_ref[...],
                            preferred_element_type=jnp.float32)
                    + b3_ref[...])


def _mlp(u_pk, i_pk, U4, V4, b1_4, W2_4, b2_4, W3_4, b3, blkp):
    bp, dpk = u_pk.shape
    h1p = U4.shape[1]
    h2p = W2_4.shape[1]
    pk = W3_4.shape[1]
    grid = (bp // blkp,)
    rep = lambda i: (0, 0)
    return pl.pallas_call(
        _mlp_body,
        grid=grid,
        in_specs=[
            pl.BlockSpec((blkp, dpk), lambda i: (i, 0)),
            pl.BlockSpec((blkp, dpk), lambda i: (i, 0)),
            pl.BlockSpec((dpk, h1p), rep),
            pl.BlockSpec((dpk, h1p), rep),
            pl.BlockSpec((1, h1p), rep),
            pl.BlockSpec((h1p, h2p), rep),
            pl.BlockSpec((1, h2p), rep),
            pl.BlockSpec((h2p, pk), rep),
            pl.BlockSpec((1, 1), rep),
        ],
        out_specs=pl.BlockSpec((blkp, pk), lambda i: (i, 0)),
        out_shape=jax.ShapeDtypeStruct((bp, pk), jnp.float32),
    )(u_pk, i_pk, U4, V4, b1_4, W2_4, b2_4, W3_4, b3)


def kernel(user_ids, content_ids, user_table, item_table, W1, b1, W2, b2, W3, b3):
    batch = user_ids.shape[0]
    d = user_table.shape[1]
    v_rows = user_table.shape[0]

    pos = jnp.arange(batch, dtype=jnp.int32)
    su, ord_u = jax.lax.sort_key_val(user_ids.astype(jnp.int32), pos)
    sc, ord_c = jax.lax.sort_key_val(content_ids.astype(jnp.int32), pos)

    u_rows = _sc_gather(su.reshape(NW, RPW // 128, 128), user_table.T,
                        d, v_rows).reshape(NW, RPW, d)
    i_rows = _sc_gather(sc.reshape(NW, RPW // 128, 128), item_table.T,
                        d, v_rows).reshape(NW, RPW, d)

    u_emb, i_emb = _sc_scatter_rows(u_rows, i_rows,
                                    ord_u.reshape(NW, RPW // 128, 128),
                                    ord_c.reshape(NW, RPW // 128, 128),
                                    batch, d)

    W1u, W1v = W1[:d, :], W1[d:, :]
    pk = 128 // d
    eye = jnp.eye(pk, dtype=jnp.float32)
    bf = jnp.bfloat16
    U4 = jnp.kron(eye, W1u).astype(bf)
    V4 = jnp.kron(eye, W1v).astype(bf)
    W2_4 = jnp.kron(eye, W2).astype(bf)
    W3_4 = jnp.kron(eye, W3).astype(bf)
    b1_4 = jnp.tile(b1, pk).reshape(1, -1)
    b2_4 = jnp.tile(b2, pk).reshape(1, -1)
    out = _mlp(u_emb.reshape(batch // pk, 128), i_emb.reshape(batch // pk, 128),
               U4, V4, b1_4, W2_4, b2_4, W3_4, b3.reshape(1, 1),
               blkp=2048)
    return out.reshape(batch, 1)
